# Initial kernel scaffold; baseline (speedup 1.0000x reference)
import jax
import jax.numpy as jnp
from jax.experimental import pallas as pl


def kernel(node1, edge1, node2, edge2, Wn, We1, be1, We2, be2, Wk, Wv, Wo, bo, ln_w, ln_b, Wr, br, seg_g1, inn_seg_i1, inn_idx_j1, out_seg_i1, out_idx_j1, seg_g2, inn_seg_i2, inn_idx_j2, out_seg_i2, out_idx_j2):
    raise NotImplementedError("write your pallas kernel here")


# speedup vs baseline: 3.7097x; 3.7097x over previous
import functools
import math

import jax
import jax.numpy as jnp
from jax import lax
from jax.experimental import pallas as pl
from jax.experimental.pallas import tpu as pltpu, tpu_sc as plsc

N = 10000
E = 320000
D = 128
D_READ = 256
STEPS = 2
B = 512
TEMP_INV = 1.0 / math.sqrt(float(D))

NC, NS = 2, 16
NW = NC * NS
EW = E // NW
C = 80
NCHUNK = EW // C
NP = 10240
RPS = NP // NS

_MESH = plsc.VectorSubcoreMesh(core_axis_name="c", subcore_axis_name="s")
_CP = pltpu.CompilerParams(needs_layout_passes=False)


def _leaky(x):
    return jnp.where(x > 0, x, 0.01 * x)


def _sc_mp_body(P_hbm, e_hbm, idx_hbm, seg_hbm, zrows_hbm, out_hbm,
                idx_v, seg_v, rows_v, e_v, msg_v, acc_sh, sem):
    c = lax.axis_index("c")
    s = lax.axis_index("s")
    w = c * NS + s
    pltpu.sync_copy(zrows_hbm, acc_sh.at[pl.ds(s * RPS, RPS)])
    plsc.subcore_barrier()

    @pl.loop(0, NCHUNK)
    def chunk(ci):
        base = w * EW + ci * C
        pltpu.sync_copy(idx_hbm.at[pl.ds(base, C)], idx_v)
        pltpu.sync_copy(seg_hbm.at[pl.ds(base, C)], seg_v)
        pltpu.async_copy(P_hbm.at[idx_v], rows_v, sem).wait()
        pltpu.sync_copy(e_hbm.at[pl.ds(base, C), :], e_v)

        @pl.loop(0, C)
        def row(r):
            for j in range(8):
                msg_v[r, pl.ds(j * 16, 16)] = (
                    rows_v[r, pl.ds(j * 16, 16)] * e_v[r, pl.ds(j * 16, 16)])

        pltpu.sync_copy(msg_v, acc_sh.at[seg_v], add=True)

    plsc.subcore_barrier()
    pltpu.sync_copy(acc_sh.at[pl.ds(s * RPS, RPS)],
                    out_hbm.at[c, pl.ds(s * RPS, RPS)])


_sc_mp = functools.partial(
    pl.kernel, _sc_mp_body, mesh=_MESH, compiler_params=_CP,
    out_type=jax.ShapeDtypeStruct((NC, NP, D), jnp.float32),
    scratch_types=[
        pltpu.VMEM((C,), jnp.int32),
        pltpu.VMEM((C,), jnp.int32),
        pltpu.VMEM((C, D), jnp.float32),
        pltpu.VMEM((C, D), jnp.float32),
        pltpu.VMEM((C, D), jnp.float32),
        pltpu.VMEM_SHARED((NP, D), jnp.float32),
        pltpu.SemaphoreType.DMA,
    ])()


def _sc_coat1_body(K_hbm, s1_hbm, s2_hbm, zn_hbm, ee_hbm, n1p_hbm, n2p_hbm,
                   s1_v, s2_v, k1_v, k2_v, t_v, ee_v, n1_v, n2_v, sem):
    c = lax.axis_index("c")
    s = lax.axis_index("s")
    w = c * NS + s
    pltpu.sync_copy(zn_hbm, n1_v)
    pltpu.sync_copy(zn_hbm, n2_v)

    @pl.loop(0, NCHUNK)
    def chunk(ci):
        base = w * EW + ci * C
        pltpu.sync_copy(s1_hbm.at[pl.ds(base, C)], s1_v)
        pltpu.sync_copy(s2_hbm.at[pl.ds(base, C)], s2_v)
        pltpu.async_copy(K_hbm.at[s1_v], k1_v, sem).wait()
        pltpu.async_copy(K_hbm.at[s2_v], k2_v, sem).wait()
        for i in range(C // 16):
            t_v[pl.ds(i * 16, 16)] = jnp.zeros((16,), jnp.float32)

        @pl.loop(0, C)
        def row(r):
            v = k1_v[r, pl.ds(0, 16)] * k2_v[r, pl.ds(0, 16)]
            for j in range(1, 8):
                v = v + k1_v[r, pl.ds(j * 16, 16)] * k2_v[r, pl.ds(j * 16, 16)]
            plsc.addupdate_scatter(t_v, [jnp.full((16,), r, jnp.int32)], v)

        for i in range(C // 16):
            ee = jnp.exp(t_v[pl.ds(i * 16, 16)] * TEMP_INV)
            ee_v[pl.ds(i * 16, 16)] = ee
            plsc.addupdate_scatter(n1_v, [s1_v[pl.ds(i * 16, 16)]], ee)
            s2raw = s2_v[pl.ds(i * 16, 16)] - jnp.full((16,), N, jnp.int32)
            plsc.addupdate_scatter(n2_v, [s2raw], ee)
        pltpu.sync_copy(ee_v, ee_hbm.at[pl.ds(base, C)])

    pltpu.sync_copy(n1_v, n1p_hbm.at[w])
    pltpu.sync_copy(n2_v, n2p_hbm.at[w])


_sc_coat1 = functools.partial(
    pl.kernel, _sc_coat1_body, mesh=_MESH, compiler_params=_CP,
    out_type=(jax.ShapeDtypeStruct((E,), jnp.float32),
              jax.ShapeDtypeStruct((NW, N), jnp.float32),
              jax.ShapeDtypeStruct((NW, N), jnp.float32)),
    scratch_types=[
        pltpu.VMEM((C,), jnp.int32),
        pltpu.VMEM((C,), jnp.int32),
        pltpu.VMEM((C, D), jnp.float32),
        pltpu.VMEM((C, D), jnp.float32),
        pltpu.VMEM((C,), jnp.float32),
        pltpu.VMEM((C,), jnp.float32),
        pltpu.VMEM((N,), jnp.float32),
        pltpu.VMEM((N,), jnp.float32),
        pltpu.SemaphoreType.DMA,
    ])()


def _sc_coat2_body(V_hbm, gidx_hbm, sidx_hbm, ee_hbm, norm_hbm, zrows_hbm,
                   a_hbm, out_hbm,
                   gidx_v, sidx_v, v_v, msg_v, ee_v, a_v, norm_v, acc_sh, sem):
    c = lax.axis_index("c")
    s = lax.axis_index("s")
    w = c * NS + s
    pltpu.sync_copy(zrows_hbm, acc_sh.at[pl.ds(s * RPS, RPS)])
    pltpu.sync_copy(norm_hbm, norm_v)
    plsc.subcore_barrier()

    @pl.loop(0, NCHUNK)
    def chunk(ci):
        base = w * EW + ci * C
        pltpu.sync_copy(gidx_hbm.at[pl.ds(base, C)], gidx_v)
        pltpu.sync_copy(sidx_hbm.at[pl.ds(base, C)], sidx_v)
        pltpu.async_copy(V_hbm.at[gidx_v], v_v, sem).wait()
        pltpu.sync_copy(ee_hbm.at[pl.ds(base, C)], ee_v)
        for i in range(C // 16):
            sv = sidx_v[pl.ds(i * 16, 16)]
            nv = plsc.load_gather(norm_v, [sv])
            av = ee_v[pl.ds(i * 16, 16)] / (nv + 1e-8)
            a_v[pl.ds(i * 16, 16)] = av
        pltpu.sync_copy(a_v, a_hbm.at[pl.ds(base, C)])

        @pl.loop(0, C)
        def row(r):
            av = plsc.load_gather(a_v, [jnp.full((16,), r, jnp.int32)])
            for j in range(8):
                msg_v[r, pl.ds(j * 16, 16)] = v_v[r, pl.ds(j * 16, 16)] * av

        pltpu.sync_copy(msg_v, acc_sh.at[sidx_v], add=True)

    plsc.subcore_barrier()
    pltpu.sync_copy(acc_sh.at[pl.ds(s * RPS, RPS)],
                    out_hbm.at[c, pl.ds(s * RPS, RPS)])


_sc_coat2 = functools.partial(
    pl.kernel, _sc_coat2_body, mesh=_MESH, compiler_params=_CP,
    out_type=(jax.ShapeDtypeStruct((E,), jnp.float32),
              jax.ShapeDtypeStruct((NC, NP, D), jnp.float32)),
    scratch_types=[
        pltpu.VMEM((C,), jnp.int32),
        pltpu.VMEM((C,), jnp.int32),
        pltpu.VMEM((C, D), jnp.float32),
        pltpu.VMEM((C, D), jnp.float32),
        pltpu.VMEM((C,), jnp.float32),
        pltpu.VMEM((C,), jnp.float32),
        pltpu.VMEM((N,), jnp.float32),
        pltpu.VMEM_SHARED((NP, D), jnp.float32),
        pltpu.SemaphoreType.DMA,
    ])()


_EBLK = 1000


def _tc_edge_body(x_ref, w1_ref, b1_ref, w2_ref, b2_ref, o0_ref, o1_ref):
    x = x_ref[...]
    for s, o_ref in ((0, o0_ref), (1, o1_ref)):
        h = _leaky(jnp.dot(x, w1_ref[s], preferred_element_type=jnp.float32)
                   + b1_ref[s])
        o_ref[...] = _leaky(
            jnp.dot(h, w2_ref[s], preferred_element_type=jnp.float32)
            + b2_ref[s])


def _tc_edge(edge, We1, be1, We2, be2):
    nblk = E // _EBLK
    return pl.pallas_call(
        _tc_edge_body,
        grid=(nblk,),
        in_specs=[
            pl.BlockSpec((_EBLK, D), lambda i: (i, 0)),
            pl.BlockSpec((STEPS, D, D), lambda i: (0, 0, 0)),
            pl.BlockSpec((STEPS, D), lambda i: (0, 0)),
            pl.BlockSpec((STEPS, D, D), lambda i: (0, 0, 0)),
            pl.BlockSpec((STEPS, D), lambda i: (0, 0)),
        ],
        out_specs=(pl.BlockSpec((_EBLK, D), lambda i: (i, 0)),
                   pl.BlockSpec((_EBLK, D), lambda i: (i, 0))),
        out_shape=(jax.ShapeDtypeStruct((E, D), jnp.float32),
                   jax.ShapeDtypeStruct((E, D), jnp.float32)),
    )(edge, We1, be1, We2, be2)


_NBLK = 1000


def _tc_proj_body(x_ref, wn_ref, wk_ref, wv_ref, p_ref, k_ref, v_ref):
    x = x_ref[...]
    p_ref[...] = jnp.dot(x, wn_ref[...], preferred_element_type=jnp.float32)
    k_ref[...] = jnp.dot(x, wk_ref[...], preferred_element_type=jnp.float32)
    v_ref[...] = jnp.dot(x, wv_ref[...], preferred_element_type=jnp.float32)


def _tc_proj(nodes, Wn, Wk, Wv):
    nblk = (2 * N) // _NBLK
    spec = pl.BlockSpec((_NBLK, D), lambda i: (i, 0))
    wspec = pl.BlockSpec((D, D), lambda i: (0, 0))
    return pl.pallas_call(
        _tc_proj_body,
        grid=(nblk,),
        in_specs=[spec, wspec, wspec, wspec],
        out_specs=(spec, spec, spec),
        out_shape=tuple(jax.ShapeDtypeStruct((2 * N, D), jnp.float32)
                        for _ in range(3)),
    )(nodes, Wn, Wk, Wv)


def _tc_normred_body(n1_ref, n2_ref, o1_ref, o2_ref):
    o1_ref[...] = jnp.sum(n1_ref[...], axis=0, keepdims=True)
    o2_ref[...] = jnp.sum(n2_ref[...], axis=0, keepdims=True)


def _tc_normred(n1p, n2p):
    o1, o2 = pl.pallas_call(
        _tc_normred_body,
        out_shape=(jax.ShapeDtypeStruct((1, N), jnp.float32),
                   jax.ShapeDtypeStruct((1, N), jnp.float32)),
    )(n1p, n2p)
    return o1.reshape(N), o2.reshape(N)


def _tc_update_body(x_ref, imp_ref, mp_ref, wo_ref, bo_ref, lw_ref, lb_ref,
                    o_ref):
    m = mp_ref[0] + mp_ref[1]
    om = _leaky(jnp.dot(m, wo_ref[...], preferred_element_type=jnp.float32)
                + bo_ref[...])
    x = x_ref[...] + imp_ref[0] + imp_ref[1] + om
    mu = jnp.mean(x, axis=-1, keepdims=True)
    var = jnp.mean((x - mu) ** 2, axis=-1, keepdims=True)
    o_ref[...] = (x - mu) / jnp.sqrt(var + 1e-5) * lw_ref[...] + lb_ref[...]


def _tc_update(node, imp, mparts, Wo, bo, lw, lb):
    nblk = N // _NBLK
    spec = pl.BlockSpec((_NBLK, D), lambda i: (i, 0))
    pspec = pl.BlockSpec((NC, _NBLK, D), lambda i: (0, i, 0))
    return pl.pallas_call(
        _tc_update_body,
        grid=(nblk,),
        in_specs=[spec, pspec, pspec,
                  pl.BlockSpec((D, D), lambda i: (0, 0)),
                  pl.BlockSpec((1, D), lambda i: (0, 0)),
                  pl.BlockSpec((1, D), lambda i: (0, 0)),
                  pl.BlockSpec((1, D), lambda i: (0, 0))],
        out_specs=spec,
        out_shape=jax.ShapeDtypeStruct((N, D), jnp.float32),
    )(node, imp, mparts, Wo, bo, lw, lb)


def _tc_readout_body(x_ref, seg_ref, wr_ref, br_ref, o_ref):
    i = pl.program_id(0)
    h = jnp.dot(x_ref[...], wr_ref[...],
                preferred_element_type=jnp.float32) + br_ref[...]
    seg = seg_ref[0]
    onehot = (lax.broadcasted_iota(jnp.int32, (2 * B, _NBLK), 0)
              == seg).astype(jnp.float32)
    g = jnp.dot(onehot, h, preferred_element_type=jnp.float32)

    @pl.when(i == 0)
    def _():
        o_ref[...] = jnp.zeros_like(o_ref)

    o_ref[...] += g


def _tc_readout(nodes, seg, Wr, br):
    nblk = (2 * N) // _NBLK
    return pl.pallas_call(
        _tc_readout_body,
        grid=(nblk,),
        in_specs=[
            pl.BlockSpec((_NBLK, D), lambda i: (i, 0)),
            pl.BlockSpec((1, 1, _NBLK), lambda i: (i, 0, 0)),
            pl.BlockSpec((D, D_READ), lambda i: (0, 0)),
            pl.BlockSpec((1, D_READ), lambda i: (0, 0)),
        ],
        out_specs=pl.BlockSpec((2 * B, D_READ), lambda i: (0, 0)),
        out_shape=jax.ShapeDtypeStruct((2 * B, D_READ), jnp.float32),
    )(nodes, seg, Wr, br)


def kernel(node1, edge1, node2, edge2, Wn, We1, be1, We2, be2, Wk, Wv, Wo, bo,
           ln_w, ln_b, Wr, br,
           seg_g1, inn_seg_i1, inn_idx_j1, out_seg_i1, out_idx_j1,
           seg_g2, inn_seg_i2, inn_idx_j2, out_seg_i2, out_idx_j2):
    f32 = jnp.float32
    zrows = jnp.zeros((RPS, D), f32)
    znorm = jnp.zeros((N,), f32)

    j2p = inn_idx_j2 + N
    s2p = out_seg_i2 + N

    e1_s = _tc_edge(edge1, We1, be1, We2, be2)
    e2_s = _tc_edge(edge2, We1, be1, We2, be2)

    nodes = jnp.concatenate([node1, node2], axis=0)
    a1 = a2 = None
    for s in range(STEPS):
        P, K, V = _tc_proj(nodes, Wn[s], Wk[s], Wv[s])
        ee, n1p, n2p = _sc_coat1(K, out_seg_i1, s2p, znorm)
        norm1, norm2 = _tc_normred(n1p, n2p)
        a1, m1p = _sc_coat2(V, s2p, out_seg_i1, ee, norm1, zrows)
        a2, m2p = _sc_coat2(V, out_seg_i1, out_seg_i2, ee, norm2, zrows)
        im1p = _sc_mp(P, e1_s[s], inn_idx_j1, inn_seg_i1, zrows)
        im2p = _sc_mp(P, e2_s[s], j2p, inn_seg_i2, zrows)
        bo_r = bo[s].reshape(1, D)
        lw_r = ln_w[s].reshape(1, D)
        lb_r = ln_b[s].reshape(1, D)
        n1 = _tc_update(nodes[:N], im1p, m1p, Wo[s], bo_r, lw_r, lb_r)
        n2 = _tc_update(nodes[N:], im2p, m2p, Wo[s], bo_r, lw_r, lb_r)
        nodes = jnp.concatenate([n1, n2], axis=0)

    seg = jnp.concatenate([seg_g1, seg_g2 + B]).reshape(2 * N // _NBLK, 1,
                                                        _NBLK)
    g = _tc_readout(nodes, seg, Wr, br.reshape(1, D_READ))
    return (g[:B], g[B:], a1[:, None], a2[:, None])


# speedup vs baseline: 8.7168x; 2.3497x over previous
import functools
import math

import jax
import jax.numpy as jnp
from jax import lax
from jax.experimental import pallas as pl
from jax.experimental.pallas import tpu as pltpu, tpu_sc as plsc

N = 10000
E = 320000
D = 128
D_READ = 256
STEPS = 2
B = 512
TEMP_INV = 1.0 / math.sqrt(float(D))

NC, NS = 2, 16
NW = NC * NS
EW = E // NW
C = 80
NCHUNK = EW // C
NP = 10240
RPS = NP // NS

_MESH = plsc.VectorSubcoreMesh(core_axis_name="c", subcore_axis_name="s")
_CP = pltpu.CompilerParams(needs_layout_passes=False)


def _leaky(x):
    return jnp.where(x > 0, x, 0.01 * x)


def _pipeline(w, issue_idx, issue_gather, compute):
    issue_idx(0, 0, wait=True)
    issue_gather(0, 0)
    issue_idx(1, 1, wait=False)

    def half(ci, cur, nxt):
        issue_idx(ci + 1, nxt, wait=True, wait_only=True)
        issue_gather(ci + 1, nxt)
        issue_gather(ci, cur, wait_only=True)
        compute(ci, cur)
        nxt_ci = jnp.minimum(ci + 2, NCHUNK - 1)
        issue_idx(nxt_ci, cur, wait=False)

    @pl.loop(0, (NCHUNK - 1) // 2)
    def pair(p):
        half(2 * p, 0, 1)
        half(2 * p + 1, 1, 0)

    issue_idx(NCHUNK - 1, 1, wait=True, wait_only=True)
    issue_gather(NCHUNK - 1, 0, wait_only=True)
    compute(NCHUNK - 1, 0)


def _sc_mp_body(P_hbm, e_hbm, idx_hbm, seg_hbm, zrows_hbm, out_hbm,
                idx0, idx1, seg0, seg1, rows0, rows1, e0, e1,
                isem0, isem1, gsem0, gsem1, acc_sh):
    c = lax.axis_index("c")
    s = lax.axis_index("s")
    w = c * NS + s
    idx_v = (idx0, idx1)
    seg_v = (seg0, seg1)
    rows_v = (rows0, rows1)
    e_v = (e0, e1)
    isem = (isem0, isem1)
    gsem = (gsem0, gsem1)
    pltpu.sync_copy(zrows_hbm, acc_sh.at[pl.ds(s * RPS, RPS)])
    plsc.subcore_barrier()

    def issue_idx(ci, b, wait, wait_only=False):
        base = w * EW + ci * C
        d1 = pltpu.make_async_copy(idx_hbm.at[pl.ds(base, C)], idx_v[b],
                                   isem[b])
        d2 = pltpu.make_async_copy(seg_hbm.at[pl.ds(base, C)], seg_v[b],
                                   isem[b])
        if not wait_only:
            d1.start()
            d2.start()
        if wait:
            d1.wait()
            d2.wait()

    def issue_gather(ci, b, wait_only=False):
        base = w * EW + ci * C
        d1 = pltpu.make_async_copy(P_hbm.at[idx_v[b]], rows_v[b], gsem[b])
        d2 = pltpu.make_async_copy(e_hbm.at[pl.ds(base, C), :], e_v[b],
                                   gsem[b])
        if wait_only:
            d1.wait()
            d2.wait()
        else:
            d1.start()
            d2.start()

    def compute(ci, b):
        rv, ev = rows_v[b], e_v[b]

        @pl.loop(0, C)
        def row(r):
            for j in range(8):
                rv[r, pl.ds(j * 16, 16)] = (
                    rv[r, pl.ds(j * 16, 16)] * ev[r, pl.ds(j * 16, 16)])

        pltpu.sync_copy(rv, acc_sh.at[seg_v[b]], add=True)

    _pipeline(w, issue_idx, issue_gather, compute)
    plsc.subcore_barrier()
    pltpu.sync_copy(acc_sh.at[pl.ds(s * RPS, RPS)],
                    out_hbm.at[c, pl.ds(s * RPS, RPS)])


_sc_mp = functools.partial(
    pl.kernel, _sc_mp_body, mesh=_MESH, compiler_params=_CP,
    out_type=jax.ShapeDtypeStruct((NC, NP, D), jnp.float32),
    scratch_types=[
        pltpu.VMEM((C,), jnp.int32), pltpu.VMEM((C,), jnp.int32),
        pltpu.VMEM((C,), jnp.int32), pltpu.VMEM((C,), jnp.int32),
        pltpu.VMEM((C, D), jnp.float32), pltpu.VMEM((C, D), jnp.float32),
        pltpu.VMEM((C, D), jnp.float32), pltpu.VMEM((C, D), jnp.float32),
        pltpu.SemaphoreType.DMA, pltpu.SemaphoreType.DMA,
        pltpu.SemaphoreType.DMA, pltpu.SemaphoreType.DMA,
        pltpu.VMEM_SHARED((NP, D), jnp.float32),
    ])()


def _sc_coat1_body(K_hbm, s1_hbm, s2_hbm, zn_hbm, ee_hbm, n1p_hbm, n2p_hbm,
                   s10, s11, s20, s21, k10, k11, k20, k21,
                   isem0, isem1, gsem0, gsem1,
                   t_v, ee_v, n1_v, n2_v):
    c = lax.axis_index("c")
    s = lax.axis_index("s")
    w = c * NS + s
    s1_v = (s10, s11)
    s2_v = (s20, s21)
    k1_v = (k10, k11)
    k2_v = (k20, k21)
    isem = (isem0, isem1)
    gsem = (gsem0, gsem1)
    pltpu.sync_copy(zn_hbm, n1_v)
    pltpu.sync_copy(zn_hbm, n2_v)

    def issue_idx(ci, b, wait, wait_only=False):
        base = w * EW + ci * C
        d1 = pltpu.make_async_copy(s1_hbm.at[pl.ds(base, C)], s1_v[b],
                                   isem[b])
        d2 = pltpu.make_async_copy(s2_hbm.at[pl.ds(base, C)], s2_v[b],
                                   isem[b])
        if not wait_only:
            d1.start()
            d2.start()
        if wait:
            d1.wait()
            d2.wait()

    def issue_gather(ci, b, wait_only=False):
        d1 = pltpu.make_async_copy(K_hbm.at[s1_v[b]], k1_v[b], gsem[b])
        d2 = pltpu.make_async_copy(K_hbm.at[s2_v[b]], k2_v[b], gsem[b])
        if wait_only:
            d1.wait()
            d2.wait()
        else:
            d1.start()
            d2.start()

    def compute(ci, b):
        base = w * EW + ci * C
        k1b, k2b = k1_v[b], k2_v[b]
        for i in range(C // 16):
            t_v[pl.ds(i * 16, 16)] = jnp.zeros((16,), jnp.float32)

        @pl.loop(0, C)
        def row(r):
            v = k1b[r, pl.ds(0, 16)] * k2b[r, pl.ds(0, 16)]
            for j in range(1, 8):
                v = v + k1b[r, pl.ds(j * 16, 16)] * k2b[r, pl.ds(j * 16, 16)]
            plsc.addupdate_scatter(t_v, [jnp.full((16,), r, jnp.int32)], v)

        for i in range(C // 16):
            ee = jnp.exp(t_v[pl.ds(i * 16, 16)] * TEMP_INV)
            ee_v[pl.ds(i * 16, 16)] = ee
            plsc.addupdate_scatter(n1_v, [s1_v[b][pl.ds(i * 16, 16)]], ee)
            s2raw = s2_v[b][pl.ds(i * 16, 16)] - jnp.full((16,), N, jnp.int32)
            plsc.addupdate_scatter(n2_v, [s2raw], ee)
        pltpu.sync_copy(ee_v, ee_hbm.at[pl.ds(base, C)])

    _pipeline(w, issue_idx, issue_gather, compute)
    pltpu.sync_copy(n1_v, n1p_hbm.at[w])
    pltpu.sync_copy(n2_v, n2p_hbm.at[w])


_sc_coat1 = functools.partial(
    pl.kernel, _sc_coat1_body, mesh=_MESH, compiler_params=_CP,
    out_type=(jax.ShapeDtypeStruct((E,), jnp.float32),
              jax.ShapeDtypeStruct((NW, N), jnp.float32),
              jax.ShapeDtypeStruct((NW, N), jnp.float32)),
    scratch_types=[
        pltpu.VMEM((C,), jnp.int32), pltpu.VMEM((C,), jnp.int32),
        pltpu.VMEM((C,), jnp.int32), pltpu.VMEM((C,), jnp.int32),
        pltpu.VMEM((C, D), jnp.float32), pltpu.VMEM((C, D), jnp.float32),
        pltpu.VMEM((C, D), jnp.float32), pltpu.VMEM((C, D), jnp.float32),
        pltpu.SemaphoreType.DMA, pltpu.SemaphoreType.DMA,
        pltpu.SemaphoreType.DMA, pltpu.SemaphoreType.DMA,
        pltpu.VMEM((C,), jnp.float32), pltpu.VMEM((C,), jnp.float32),
        pltpu.VMEM((N,), jnp.float32), pltpu.VMEM((N,), jnp.float32),
    ])()


def _sc_vscatter_body(V_hbm, gidx_hbm, sidx_hbm, ee_hbm, zrows_hbm, out_hbm,
                      gidx0, gidx1, sidx0, sidx1, v0, v1, ee0, ee1,
                      isem0, isem1, gsem0, gsem1, acc_sh):
    c = lax.axis_index("c")
    s = lax.axis_index("s")
    w = c * NS + s
    gidx_v = (gidx0, gidx1)
    sidx_v = (sidx0, sidx1)
    v_v = (v0, v1)
    ee_v = (ee0, ee1)
    isem = (isem0, isem1)
    gsem = (gsem0, gsem1)
    pltpu.sync_copy(zrows_hbm, acc_sh.at[pl.ds(s * RPS, RPS)])
    plsc.subcore_barrier()

    def issue_idx(ci, b, wait, wait_only=False):
        base = w * EW + ci * C
        d1 = pltpu.make_async_copy(gidx_hbm.at[pl.ds(base, C)], gidx_v[b],
                                   isem[b])
        d2 = pltpu.make_async_copy(sidx_hbm.at[pl.ds(base, C)], sidx_v[b],
                                   isem[b])
        if not wait_only:
            d1.start()
            d2.start()
        if wait:
            d1.wait()
            d2.wait()

    def issue_gather(ci, b, wait_only=False):
        base = w * EW + ci * C
        d1 = pltpu.make_async_copy(V_hbm.at[gidx_v[b]], v_v[b], gsem[b])
        d2 = pltpu.make_async_copy(ee_hbm.at[pl.ds(base, C)], ee_v[b],
                                   gsem[b])
        if wait_only:
            d1.wait()
            d2.wait()
        else:
            d1.start()
            d2.start()

    def compute(ci, b):
        vb, eb = v_v[b], ee_v[b]

        @pl.loop(0, C)
        def row(r):
            av = plsc.load_gather(eb, [jnp.full((16,), r, jnp.int32)])
            for j in range(8):
                vb[r, pl.ds(j * 16, 16)] = vb[r, pl.ds(j * 16, 16)] * av

        pltpu.sync_copy(vb, acc_sh.at[sidx_v[b]], add=True)

    _pipeline(w, issue_idx, issue_gather, compute)
    plsc.subcore_barrier()
    pltpu.sync_copy(acc_sh.at[pl.ds(s * RPS, RPS)],
                    out_hbm.at[c, pl.ds(s * RPS, RPS)])


_sc_vscatter = functools.partial(
    pl.kernel, _sc_vscatter_body, mesh=_MESH, compiler_params=_CP,
    out_type=jax.ShapeDtypeStruct((NC, NP, D), jnp.float32),
    scratch_types=[
        pltpu.VMEM((C,), jnp.int32), pltpu.VMEM((C,), jnp.int32),
        pltpu.VMEM((C,), jnp.int32), pltpu.VMEM((C,), jnp.int32),
        pltpu.VMEM((C, D), jnp.float32), pltpu.VMEM((C, D), jnp.float32),
        pltpu.VMEM((C,), jnp.float32), pltpu.VMEM((C,), jnp.float32),
        pltpu.SemaphoreType.DMA, pltpu.SemaphoreType.DMA,
        pltpu.SemaphoreType.DMA, pltpu.SemaphoreType.DMA,
        pltpu.VMEM_SHARED((NP, D), jnp.float32),
    ])()


_CA = 2000


def _sc_att_body(s1_hbm, s2_hbm, ee_hbm, n1_hbm, n2_hbm, a1_hbm, a2_hbm,
                 s1_v, s2_v, ee_v, a1_v, a2_v, n1_v, n2_v):
    c = lax.axis_index("c")
    s = lax.axis_index("s")
    w = c * NS + s
    pltpu.sync_copy(n1_hbm, n1_v)
    pltpu.sync_copy(n2_hbm, n2_v)

    @pl.loop(0, EW // _CA)
    def chunk(ci):
        base = w * EW + ci * _CA
        pltpu.sync_copy(s1_hbm.at[pl.ds(base, _CA)], s1_v)
        pltpu.sync_copy(s2_hbm.at[pl.ds(base, _CA)], s2_v)
        pltpu.sync_copy(ee_hbm.at[pl.ds(base, _CA)], ee_v)

        @pl.loop(0, _CA // 16)
        def grp(i):
            ee = ee_v[pl.ds(i * 16, 16)]
            nv1 = plsc.load_gather(n1_v, [s1_v[pl.ds(i * 16, 16)]])
            a1_v[pl.ds(i * 16, 16)] = ee / (nv1 + 1e-8)
            nv2 = plsc.load_gather(n2_v, [s2_v[pl.ds(i * 16, 16)]])
            a2_v[pl.ds(i * 16, 16)] = ee / (nv2 + 1e-8)

        pltpu.sync_copy(a1_v, a1_hbm.at[pl.ds(base, _CA)])
        pltpu.sync_copy(a2_v, a2_hbm.at[pl.ds(base, _CA)])


_sc_att = functools.partial(
    pl.kernel, _sc_att_body, mesh=_MESH, compiler_params=_CP,
    out_type=(jax.ShapeDtypeStruct((E,), jnp.float32),
              jax.ShapeDtypeStruct((E,), jnp.float32)),
    scratch_types=[
        pltpu.VMEM((_CA,), jnp.int32), pltpu.VMEM((_CA,), jnp.int32),
        pltpu.VMEM((_CA,), jnp.float32),
        pltpu.VMEM((_CA,), jnp.float32), pltpu.VMEM((_CA,), jnp.float32),
        pltpu.VMEM((N,), jnp.float32), pltpu.VMEM((N,), jnp.float32),
    ])()


_EBLK = 1000


def _tc_edge_body(x_ref, w1_ref, b1_ref, w2_ref, b2_ref, o0_ref, o1_ref):
    x = x_ref[...]
    for s, o_ref in ((0, o0_ref), (1, o1_ref)):
        h = _leaky(jnp.dot(x, w1_ref[s], preferred_element_type=jnp.float32)
                   + b1_ref[s])
        o_ref[...] = _leaky(
            jnp.dot(h, w2_ref[s], preferred_element_type=jnp.float32)
            + b2_ref[s])


def _tc_edge(edge, We1, be1, We2, be2):
    nblk = E // _EBLK
    return pl.pallas_call(
        _tc_edge_body,
        grid=(nblk,),
        in_specs=[
            pl.BlockSpec((_EBLK, D), lambda i: (i, 0)),
            pl.BlockSpec((STEPS, D, D), lambda i: (0, 0, 0)),
            pl.BlockSpec((STEPS, D), lambda i: (0, 0)),
            pl.BlockSpec((STEPS, D, D), lambda i: (0, 0, 0)),
            pl.BlockSpec((STEPS, D), lambda i: (0, 0)),
        ],
        out_specs=(pl.BlockSpec((_EBLK, D), lambda i: (i, 0)),
                   pl.BlockSpec((_EBLK, D), lambda i: (i, 0))),
        out_shape=(jax.ShapeDtypeStruct((E, D), jnp.float32),
                   jax.ShapeDtypeStruct((E, D), jnp.float32)),
    )(edge, We1, be1, We2, be2)


_NBLK = 1000


def _tc_proj_body(x_ref, wn_ref, wk_ref, wv_ref, p_ref, k_ref, v_ref):
    x = x_ref[...]
    p_ref[...] = jnp.dot(x, wn_ref[...], preferred_element_type=jnp.float32)
    k_ref[...] = jnp.dot(x, wk_ref[...], preferred_element_type=jnp.float32)
    v_ref[...] = jnp.dot(x, wv_ref[...], preferred_element_type=jnp.float32)


def _tc_proj(nodes, Wn, Wk, Wv):
    nblk = (2 * N) // _NBLK
    spec = pl.BlockSpec((_NBLK, D), lambda i: (i, 0))
    wspec = pl.BlockSpec((D, D), lambda i: (0, 0))
    return pl.pallas_call(
        _tc_proj_body,
        grid=(nblk,),
        in_specs=[spec, wspec, wspec, wspec],
        out_specs=(spec, spec, spec),
        out_shape=tuple(jax.ShapeDtypeStruct((2 * N, D), jnp.float32)
                        for _ in range(3)),
    )(nodes, Wn, Wk, Wv)


def _tc_normred_body(n1_ref, n2_ref, o1_ref, o2_ref):
    o1_ref[...] = jnp.sum(n1_ref[...], axis=0, keepdims=True)
    o2_ref[...] = jnp.sum(n2_ref[...], axis=0, keepdims=True)


def _tc_normred(n1p, n2p):
    o1, o2 = pl.pallas_call(
        _tc_normred_body,
        out_shape=(jax.ShapeDtypeStruct((1, N), jnp.float32),
                   jax.ShapeDtypeStruct((1, N), jnp.float32)),
    )(n1p, n2p)
    return o1.reshape(N), o2.reshape(N)


def _tc_update_body(x_ref, imp_ref, mp_ref, np_ref, wo_ref, bo_ref,
                    lw_ref, lb_ref, o_ref):
    norm = np_ref[...] + 1e-8
    m = (mp_ref[0] + mp_ref[1]) / norm
    om = _leaky(jnp.dot(m, wo_ref[...], preferred_element_type=jnp.float32)
                + bo_ref[...])
    x = x_ref[...] + imp_ref[0] + imp_ref[1] + om
    mu = jnp.mean(x, axis=-1, keepdims=True)
    var = jnp.mean((x - mu) ** 2, axis=-1, keepdims=True)
    o_ref[...] = (x - mu) / jnp.sqrt(var + 1e-5) * lw_ref[...] + lb_ref[...]


def _tc_update(node, imp, mparts, ncol, Wo, bo, lw, lb):
    nblk = N // _NBLK
    spec = pl.BlockSpec((_NBLK, D), lambda i: (i, 0))
    pspec = pl.BlockSpec((NC, _NBLK, D), lambda i: (0, i, 0))
    return pl.pallas_call(
        _tc_update_body,
        grid=(nblk,),
        in_specs=[spec, pspec, pspec,
                  pl.BlockSpec((_NBLK, 1), lambda i: (i, 0)),
                  pl.BlockSpec((D, D), lambda i: (0, 0)),
                  pl.BlockSpec((1, D), lambda i: (0, 0)),
                  pl.BlockSpec((1, D), lambda i: (0, 0)),
                  pl.BlockSpec((1, D), lambda i: (0, 0))],
        out_specs=spec,
        out_shape=jax.ShapeDtypeStruct((N, D), jnp.float32),
    )(node, imp, mparts, ncol, Wo, bo, lw, lb)


def _tc_readout_body(x_ref, seg_ref, wr_ref, br_ref, o_ref):
    i = pl.program_id(0)
    h = jnp.dot(x_ref[...], wr_ref[...],
                preferred_element_type=jnp.float32) + br_ref[...]
    seg = seg_ref[0]
    onehot = (lax.broadcasted_iota(jnp.int32, (2 * B, _NBLK), 0)
              == seg).astype(jnp.float32)
    g = jnp.dot(onehot, h, preferred_element_type=jnp.float32)

    @pl.when(i == 0)
    def _():
        o_ref[...] = jnp.zeros_like(o_ref)

    o_ref[...] += g


def _tc_readout(nodes, seg, Wr, br):
    nblk = (2 * N) // _NBLK
    return pl.pallas_call(
        _tc_readout_body,
        grid=(nblk,),
        in_specs=[
            pl.BlockSpec((_NBLK, D), lambda i: (i, 0)),
            pl.BlockSpec((1, 1, _NBLK), lambda i: (i, 0, 0)),
            pl.BlockSpec((D, D_READ), lambda i: (0, 0)),
            pl.BlockSpec((1, D_READ), lambda i: (0, 0)),
        ],
        out_specs=pl.BlockSpec((2 * B, D_READ), lambda i: (0, 0)),
        out_shape=jax.ShapeDtypeStruct((2 * B, D_READ), jnp.float32),
    )(nodes, seg, Wr, br)


def kernel(node1, edge1, node2, edge2, Wn, We1, be1, We2, be2, Wk, Wv, Wo, bo,
           ln_w, ln_b, Wr, br,
           seg_g1, inn_seg_i1, inn_idx_j1, out_seg_i1, out_idx_j1,
           seg_g2, inn_seg_i2, inn_idx_j2, out_seg_i2, out_idx_j2):
    f32 = jnp.float32
    zrows = jnp.zeros((RPS, D), f32)
    znorm = jnp.zeros((N,), f32)

    j2p = inn_idx_j2 + N
    s2p = out_seg_i2 + N

    e1_s = _tc_edge(edge1, We1, be1, We2, be2)
    e2_s = _tc_edge(edge2, We1, be1, We2, be2)

    nodes = jnp.concatenate([node1, node2], axis=0)
    ee = n1p = n2p = None
    for s in range(STEPS):
        P, K, V = _tc_proj(nodes, Wn[s], Wk[s], Wv[s])
        ee, n1p, n2p = _sc_coat1(K, out_seg_i1, s2p, znorm)
        m1p = _sc_vscatter(V, s2p, out_seg_i1, ee, zrows)
        m2p = _sc_vscatter(V, out_seg_i1, out_seg_i2, ee, zrows)
        im1p = _sc_mp(P, e1_s[s], inn_idx_j1, inn_seg_i1, zrows)
        im2p = _sc_mp(P, e2_s[s], j2p, inn_seg_i2, zrows)
        norm1, norm2 = _tc_normred(n1p, n2p)
        bo_r = bo[s].reshape(1, D)
        lw_r = ln_w[s].reshape(1, D)
        lb_r = ln_b[s].reshape(1, D)
        n1 = _tc_update(nodes[:N], im1p, m1p, norm1.reshape(N, 1),
                        Wo[s], bo_r, lw_r, lb_r)
        n2 = _tc_update(nodes[N:], im2p, m2p, norm2.reshape(N, 1),
                        Wo[s], bo_r, lw_r, lb_r)
        nodes = jnp.concatenate([n1, n2], axis=0)

    a1, a2 = _sc_att(out_seg_i1, out_seg_i2, ee, norm1, norm2)

    seg = jnp.concatenate([seg_g1, seg_g2 + B]).reshape(2 * N // _NBLK, 1,
                                                        _NBLK)
    g = _tc_readout(nodes, seg, Wr, br.reshape(1, D_READ))
    return (g[:B], g[B:], a1[:, None], a2[:, None])


# speedup vs baseline: 9.3911x; 1.0774x over previous
import functools
import math

import jax
import jax.numpy as jnp
from jax import lax
from jax.experimental import pallas as pl
from jax.experimental.pallas import tpu as pltpu, tpu_sc as plsc

N = 10000
E = 320000
D = 128
D_READ = 256
STEPS = 2
B = 512
TEMP_INV = 1.0 / math.sqrt(float(D))

NC, NS = 2, 16
NW = NC * NS
EW = E // NW
C = 80
NCHUNK = EW // C
NP = 10240
RPS = NP // NS

_MESH = plsc.VectorSubcoreMesh(core_axis_name="c", subcore_axis_name="s")
_CP = pltpu.CompilerParams(needs_layout_passes=False)


def _leaky(x):
    return jnp.where(x > 0, x, 0.01 * x)


def _pipeline(w, issue_idx, issue_gather, compute):
    issue_idx(0, 0, wait=True)
    issue_gather(0, 0)
    issue_idx(1, 1, wait=False)

    def half(ci, cur, nxt):
        issue_idx(ci + 1, nxt, wait=True, wait_only=True)
        issue_gather(ci + 1, nxt)
        issue_gather(ci, cur, wait_only=True)
        compute(ci, cur)
        nxt_ci = jnp.minimum(ci + 2, NCHUNK - 1)
        issue_idx(nxt_ci, cur, wait=False)

    @pl.loop(0, (NCHUNK - 1) // 2)
    def pair(p):
        half(2 * p, 0, 1)
        half(2 * p + 1, 1, 0)

    issue_idx(NCHUNK - 1, 1, wait=True, wait_only=True)
    issue_gather(NCHUNK - 1, 0, wait_only=True)
    compute(NCHUNK - 1, 0)


def _sc_mp_body(P_hbm, e_hbm, idx_hbm, seg_hbm, zrows_hbm, out_hbm,
                idx0, idx1, seg0, seg1, rows0, rows1, e0, e1,
                sseg0, sseg1,
                isem0, isem1, gsem0, gsem1, ssem0, ssem1, acc_sh):
    c = lax.axis_index("c")
    s = lax.axis_index("s")
    w = c * NS + s
    idx_v = (idx0, idx1)
    seg_v = (seg0, seg1)
    rows_v = (rows0, rows1)
    e_v = (e0, e1)
    sseg_v = (sseg0, sseg1)
    isem = (isem0, isem1)
    gsem = (gsem0, gsem1)
    ssem = (ssem0, ssem1)
    pltpu.sync_copy(zrows_hbm, acc_sh.at[pl.ds(s * RPS, RPS)])
    plsc.subcore_barrier()

    H = C // 2

    def issue_idx(ci, b, wait, wait_only=False):
        base = w * EW + ci * C
        d1 = pltpu.make_async_copy(idx_hbm.at[pl.ds(base, C)], idx_v[b],
                                   isem[b])
        d2 = pltpu.make_async_copy(seg_hbm.at[pl.ds(base, C)], seg_v[b],
                                   isem[b])
        if not wait_only:
            d1.start()
            d2.start()
        if wait:
            d1.wait()
            d2.wait()

    def issue_gather(ci, b, wait_only=False):
        base = w * EW + ci * C
        ds = [pltpu.make_async_copy(P_hbm.at[idx_v[b].at[pl.ds(h * H, H)]],
                                    rows_v[b].at[pl.ds(h * H, H)], gsem[b])
              for h in (0, 1)]
        ds.append(pltpu.make_async_copy(e_hbm.at[pl.ds(base, C), :], e_v[b],
                                        gsem[b]))
        if not wait_only:
            @pl.when(ci >= 2)
            def _():
                pltpu.make_async_copy(rows_v[b], acc_sh.at[sseg_v[b]],
                                      ssem[b]).wait()

        for d in ds:
            if wait_only:
                d.wait()
            else:
                d.start()

    def compute(ci, b):
        rv, ev, sv = rows_v[b], e_v[b], sseg_v[b]
        for i in range(C // 16):
            sv[pl.ds(i * 16, 16)] = seg_v[b][pl.ds(i * 16, 16)]

        @pl.loop(0, C)
        def row(r):
            for j in range(8):
                rv[r, pl.ds(j * 16, 16)] = (
                    rv[r, pl.ds(j * 16, 16)] * ev[r, pl.ds(j * 16, 16)])

        pltpu.async_copy(rv, acc_sh.at[sv], ssem[b], add=True)

    _pipeline(w, issue_idx, issue_gather, compute)
    pltpu.make_async_copy(rows_v[1], acc_sh.at[sseg_v[1]], ssem[1]).wait()
    pltpu.make_async_copy(rows_v[0], acc_sh.at[sseg_v[0]], ssem[0]).wait()
    plsc.subcore_barrier()
    pltpu.sync_copy(acc_sh.at[pl.ds(s * RPS, RPS)],
                    out_hbm.at[c, pl.ds(s * RPS, RPS)])


_sc_mp = functools.partial(
    pl.kernel, _sc_mp_body, mesh=_MESH, compiler_params=_CP,
    out_type=jax.ShapeDtypeStruct((NC, NP, D), jnp.float32),
    scratch_types=[
        pltpu.VMEM((C,), jnp.int32), pltpu.VMEM((C,), jnp.int32),
        pltpu.VMEM((C,), jnp.int32), pltpu.VMEM((C,), jnp.int32),
        pltpu.VMEM((C, D), jnp.float32), pltpu.VMEM((C, D), jnp.float32),
        pltpu.VMEM((C, D), jnp.float32), pltpu.VMEM((C, D), jnp.float32),
        pltpu.VMEM((C,), jnp.int32), pltpu.VMEM((C,), jnp.int32),
        pltpu.SemaphoreType.DMA, pltpu.SemaphoreType.DMA,
        pltpu.SemaphoreType.DMA, pltpu.SemaphoreType.DMA,
        pltpu.SemaphoreType.DMA, pltpu.SemaphoreType.DMA,
        pltpu.VMEM_SHARED((NP, D), jnp.float32),
    ])()


def _sc_coat1_body(K1_hbm, K2_hbm, s1_hbm, s2_hbm, zn_hbm,
                   ee_hbm, n1p_hbm, n2p_hbm,
                   s10, s11, s20, s21, k10, k11, k20, k21,
                   isem0, isem1, gsem0, gsem1,
                   t_v, ee_v, n1_v, n2_v):
    c = lax.axis_index("c")
    s = lax.axis_index("s")
    w = c * NS + s
    s1_v = (s10, s11)
    s2_v = (s20, s21)
    k1_v = (k10, k11)
    k2_v = (k20, k21)
    isem = (isem0, isem1)
    gsem = (gsem0, gsem1)
    pltpu.sync_copy(zn_hbm, n1_v)
    pltpu.sync_copy(zn_hbm, n2_v)

    def issue_idx(ci, b, wait, wait_only=False):
        base = w * EW + ci * C
        d1 = pltpu.make_async_copy(s1_hbm.at[pl.ds(base, C)], s1_v[b],
                                   isem[b])
        d2 = pltpu.make_async_copy(s2_hbm.at[pl.ds(base, C)], s2_v[b],
                                   isem[b])
        if not wait_only:
            d1.start()
            d2.start()
        if wait:
            d1.wait()
            d2.wait()

    def issue_gather(ci, b, wait_only=False):
        d1 = pltpu.make_async_copy(K1_hbm.at[s1_v[b]], k1_v[b], gsem[b])
        d2 = pltpu.make_async_copy(K2_hbm.at[s2_v[b]], k2_v[b], gsem[b])
        if wait_only:
            d1.wait()
            d2.wait()
        else:
            d1.start()
            d2.start()

    def compute(ci, b):
        base = w * EW + ci * C
        k1b, k2b = k1_v[b], k2_v[b]
        for i in range(C // 16):
            t_v[pl.ds(i * 16, 16)] = jnp.zeros((16,), jnp.float32)

        @pl.loop(0, C)
        def row(r):
            v = k1b[r, pl.ds(0, 16)] * k2b[r, pl.ds(0, 16)]
            for j in range(1, 8):
                v = v + k1b[r, pl.ds(j * 16, 16)] * k2b[r, pl.ds(j * 16, 16)]
            plsc.addupdate_scatter(t_v, [jnp.full((16,), r, jnp.int32)], v)

        for i in range(C // 16):
            ee = jnp.exp(t_v[pl.ds(i * 16, 16)] * TEMP_INV)
            ee_v[pl.ds(i * 16, 16)] = ee
            plsc.addupdate_scatter(n1_v, [s1_v[b][pl.ds(i * 16, 16)]], ee)
            plsc.addupdate_scatter(n2_v, [s2_v[b][pl.ds(i * 16, 16)]], ee)
        pltpu.sync_copy(ee_v, ee_hbm.at[pl.ds(base, C)])

    _pipeline(w, issue_idx, issue_gather, compute)
    pltpu.sync_copy(n1_v, n1p_hbm.at[w])
    pltpu.sync_copy(n2_v, n2p_hbm.at[w])


_sc_coat1 = functools.partial(
    pl.kernel, _sc_coat1_body, mesh=_MESH, compiler_params=_CP,
    out_type=(jax.ShapeDtypeStruct((E,), jnp.float32),
              jax.ShapeDtypeStruct((NW, N), jnp.float32),
              jax.ShapeDtypeStruct((NW, N), jnp.float32)),
    scratch_types=[
        pltpu.VMEM((C,), jnp.int32), pltpu.VMEM((C,), jnp.int32),
        pltpu.VMEM((C,), jnp.int32), pltpu.VMEM((C,), jnp.int32),
        pltpu.VMEM((C, D), jnp.float32), pltpu.VMEM((C, D), jnp.float32),
        pltpu.VMEM((C, D), jnp.float32), pltpu.VMEM((C, D), jnp.float32),
        pltpu.SemaphoreType.DMA, pltpu.SemaphoreType.DMA,
        pltpu.SemaphoreType.DMA, pltpu.SemaphoreType.DMA,
        pltpu.VMEM((C,), jnp.float32), pltpu.VMEM((C,), jnp.float32),
        pltpu.VMEM((N,), jnp.float32), pltpu.VMEM((N,), jnp.float32),
    ])()


def _sc_vscatter_body(V_hbm, gidx_hbm, sidx_hbm, ee_hbm, zrows_hbm, out_hbm,
                      gidx0, gidx1, sidx0, sidx1, v0, v1, ee0, ee1,
                      ssidx0, ssidx1,
                      isem0, isem1, gsem0, gsem1, ssem0, ssem1, acc_sh):
    c = lax.axis_index("c")
    s = lax.axis_index("s")
    w = c * NS + s
    gidx_v = (gidx0, gidx1)
    sidx_v = (sidx0, sidx1)
    v_v = (v0, v1)
    ee_v = (ee0, ee1)
    ssidx_v = (ssidx0, ssidx1)
    isem = (isem0, isem1)
    gsem = (gsem0, gsem1)
    ssem = (ssem0, ssem1)
    pltpu.sync_copy(zrows_hbm, acc_sh.at[pl.ds(s * RPS, RPS)])
    plsc.subcore_barrier()

    H = C // 2

    def issue_idx(ci, b, wait, wait_only=False):
        base = w * EW + ci * C
        d1 = pltpu.make_async_copy(gidx_hbm.at[pl.ds(base, C)], gidx_v[b],
                                   isem[b])
        d2 = pltpu.make_async_copy(sidx_hbm.at[pl.ds(base, C)], sidx_v[b],
                                   isem[b])
        if not wait_only:
            d1.start()
            d2.start()
        if wait:
            d1.wait()
            d2.wait()

    def issue_gather(ci, b, wait_only=False):
        base = w * EW + ci * C
        ds = [pltpu.make_async_copy(V_hbm.at[gidx_v[b].at[pl.ds(h * H, H)]],
                                    v_v[b].at[pl.ds(h * H, H)], gsem[b])
              for h in (0, 1)]
        ds.append(pltpu.make_async_copy(ee_hbm.at[pl.ds(base, C)], ee_v[b],
                                        gsem[b]))
        if not wait_only:
            @pl.when(ci >= 2)
            def _():
                pltpu.make_async_copy(v_v[b], acc_sh.at[ssidx_v[b]],
                                      ssem[b]).wait()

        for d in ds:
            if wait_only:
                d.wait()
            else:
                d.start()

    def compute(ci, b):
        vb, eb, sv = v_v[b], ee_v[b], ssidx_v[b]
        for i in range(C // 16):
            sv[pl.ds(i * 16, 16)] = sidx_v[b][pl.ds(i * 16, 16)]

        @pl.loop(0, C)
        def row(r):
            av = plsc.load_gather(eb, [jnp.full((16,), r, jnp.int32)])
            for j in range(8):
                vb[r, pl.ds(j * 16, 16)] = vb[r, pl.ds(j * 16, 16)] * av

        pltpu.async_copy(vb, acc_sh.at[sv], ssem[b], add=True)

    _pipeline(w, issue_idx, issue_gather, compute)
    pltpu.make_async_copy(v_v[1], acc_sh.at[ssidx_v[1]], ssem[1]).wait()
    pltpu.make_async_copy(v_v[0], acc_sh.at[ssidx_v[0]], ssem[0]).wait()
    plsc.subcore_barrier()
    pltpu.sync_copy(acc_sh.at[pl.ds(s * RPS, RPS)],
                    out_hbm.at[c, pl.ds(s * RPS, RPS)])


_sc_vscatter = functools.partial(
    pl.kernel, _sc_vscatter_body, mesh=_MESH, compiler_params=_CP,
    out_type=jax.ShapeDtypeStruct((NC, NP, D), jnp.float32),
    scratch_types=[
        pltpu.VMEM((C,), jnp.int32), pltpu.VMEM((C,), jnp.int32),
        pltpu.VMEM((C,), jnp.int32), pltpu.VMEM((C,), jnp.int32),
        pltpu.VMEM((C, D), jnp.float32), pltpu.VMEM((C, D), jnp.float32),
        pltpu.VMEM((C,), jnp.float32), pltpu.VMEM((C,), jnp.float32),
        pltpu.VMEM((C,), jnp.int32), pltpu.VMEM((C,), jnp.int32),
        pltpu.SemaphoreType.DMA, pltpu.SemaphoreType.DMA,
        pltpu.SemaphoreType.DMA, pltpu.SemaphoreType.DMA,
        pltpu.SemaphoreType.DMA, pltpu.SemaphoreType.DMA,
        pltpu.VMEM_SHARED((NP, D), jnp.float32),
    ])()


_CA = 2000


def _sc_att_body(s1_hbm, s2_hbm, ee_hbm, n1_hbm, n2_hbm, a1_hbm, a2_hbm,
                 s1_v, s2_v, ee_v, a1_v, a2_v, n1_v, n2_v):
    c = lax.axis_index("c")
    s = lax.axis_index("s")
    w = c * NS + s
    pltpu.sync_copy(n1_hbm, n1_v)
    pltpu.sync_copy(n2_hbm, n2_v)

    @pl.loop(0, EW // _CA)
    def chunk(ci):
        base = w * EW + ci * _CA
        pltpu.sync_copy(s1_hbm.at[pl.ds(base, _CA)], s1_v)
        pltpu.sync_copy(s2_hbm.at[pl.ds(base, _CA)], s2_v)
        pltpu.sync_copy(ee_hbm.at[pl.ds(base, _CA)], ee_v)

        @pl.loop(0, _CA // 16)
        def grp(i):
            ee = ee_v[pl.ds(i * 16, 16)]
            nv1 = plsc.load_gather(n1_v, [s1_v[pl.ds(i * 16, 16)]])
            a1_v[pl.ds(i * 16, 16)] = ee / (nv1 + 1e-8)
            nv2 = plsc.load_gather(n2_v, [s2_v[pl.ds(i * 16, 16)]])
            a2_v[pl.ds(i * 16, 16)] = ee / (nv2 + 1e-8)

        pltpu.sync_copy(a1_v, a1_hbm.at[pl.ds(base, _CA)])
        pltpu.sync_copy(a2_v, a2_hbm.at[pl.ds(base, _CA)])


_sc_att = functools.partial(
    pl.kernel, _sc_att_body, mesh=_MESH, compiler_params=_CP,
    out_type=(jax.ShapeDtypeStruct((E,), jnp.float32),
              jax.ShapeDtypeStruct((E,), jnp.float32)),
    scratch_types=[
        pltpu.VMEM((_CA,), jnp.int32), pltpu.VMEM((_CA,), jnp.int32),
        pltpu.VMEM((_CA,), jnp.float32),
        pltpu.VMEM((_CA,), jnp.float32), pltpu.VMEM((_CA,), jnp.float32),
        pltpu.VMEM((N,), jnp.float32), pltpu.VMEM((N,), jnp.float32),
    ])()


_EBLK = 1000


def _tc_edge_body(x_ref, w1_ref, b1_ref, w2_ref, b2_ref, o0_ref, o1_ref):
    bf = jnp.bfloat16
    x = x_ref[...].astype(bf)
    for s, o_ref in ((0, o0_ref), (1, o1_ref)):
        h = _leaky(jnp.dot(x, w1_ref[s].astype(bf),
                           preferred_element_type=jnp.float32) + b1_ref[s])
        o_ref[...] = _leaky(
            jnp.dot(h.astype(bf), w2_ref[s].astype(bf),
                    preferred_element_type=jnp.float32)
            + b2_ref[s])


def _tc_edge(edge, We1, be1, We2, be2):
    nblk = E // _EBLK
    return pl.pallas_call(
        _tc_edge_body,
        grid=(nblk,),
        in_specs=[
            pl.BlockSpec((_EBLK, D), lambda i: (i, 0)),
            pl.BlockSpec((STEPS, D, D), lambda i: (0, 0, 0)),
            pl.BlockSpec((STEPS, D), lambda i: (0, 0)),
            pl.BlockSpec((STEPS, D, D), lambda i: (0, 0, 0)),
            pl.BlockSpec((STEPS, D), lambda i: (0, 0)),
        ],
        out_specs=(pl.BlockSpec((_EBLK, D), lambda i: (i, 0)),
                   pl.BlockSpec((_EBLK, D), lambda i: (i, 0))),
        out_shape=(jax.ShapeDtypeStruct((E, D), jnp.float32),
                   jax.ShapeDtypeStruct((E, D), jnp.float32)),
    )(edge, We1, be1, We2, be2)


_NBLK = 1000


def _tc_proj_body(x_ref, wn_ref, wk_ref, wv_ref, p_ref, k_ref, v_ref):
    x = x_ref[...]
    p_ref[...] = jnp.dot(x, wn_ref[...], preferred_element_type=jnp.float32)
    k_ref[...] = jnp.dot(x, wk_ref[...], preferred_element_type=jnp.float32)
    v_ref[...] = jnp.dot(x, wv_ref[...], preferred_element_type=jnp.float32)


def _tc_proj(nodes, Wn, Wk, Wv):
    nblk = N // _NBLK
    spec = pl.BlockSpec((_NBLK, D), lambda i: (i, 0))
    wspec = pl.BlockSpec((D, D), lambda i: (0, 0))
    return pl.pallas_call(
        _tc_proj_body,
        grid=(nblk,),
        in_specs=[spec, wspec, wspec, wspec],
        out_specs=(spec, spec, spec),
        out_shape=tuple(jax.ShapeDtypeStruct((N, D), jnp.float32)
                        for _ in range(3)),
    )(nodes, Wn, Wk, Wv)


def _tc_normred_body(n1_ref, n2_ref, o1_ref, o2_ref):
    o1_ref[...] = jnp.sum(n1_ref[...], axis=0, keepdims=True)
    o2_ref[...] = jnp.sum(n2_ref[...], axis=0, keepdims=True)


def _tc_normred(n1p, n2p):
    o1, o2 = pl.pallas_call(
        _tc_normred_body,
        out_shape=(jax.ShapeDtypeStruct((1, N), jnp.float32),
                   jax.ShapeDtypeStruct((1, N), jnp.float32)),
    )(n1p, n2p)
    return o1.reshape(N), o2.reshape(N)


def _tc_update_body(x_ref, imp_ref, mp_ref, np_ref, wo_ref, bo_ref,
                    lw_ref, lb_ref, o_ref):
    norm = np_ref[...] + 1e-8
    m = (mp_ref[0] + mp_ref[1]) / norm
    om = _leaky(jnp.dot(m, wo_ref[...], preferred_element_type=jnp.float32)
                + bo_ref[...])
    x = x_ref[...] + imp_ref[0] + imp_ref[1] + om
    mu = jnp.mean(x, axis=-1, keepdims=True)
    var = jnp.mean((x - mu) ** 2, axis=-1, keepdims=True)
    o_ref[...] = (x - mu) / jnp.sqrt(var + 1e-5) * lw_ref[...] + lb_ref[...]


def _tc_update(node, imp, mparts, ncol, Wo, bo, lw, lb):
    nblk = N // _NBLK
    spec = pl.BlockSpec((_NBLK, D), lambda i: (i, 0))
    pspec = pl.BlockSpec((NC, _NBLK, D), lambda i: (0, i, 0))
    return pl.pallas_call(
        _tc_update_body,
        grid=(nblk,),
        in_specs=[spec, pspec, pspec,
                  pl.BlockSpec((_NBLK, 1), lambda i: (i, 0)),
                  pl.BlockSpec((D, D), lambda i: (0, 0)),
                  pl.BlockSpec((1, D), lambda i: (0, 0)),
                  pl.BlockSpec((1, D), lambda i: (0, 0)),
                  pl.BlockSpec((1, D), lambda i: (0, 0))],
        out_specs=spec,
        out_shape=jax.ShapeDtypeStruct((N, D), jnp.float32),
    )(node, imp, mparts, ncol, Wo, bo, lw, lb)


def _tc_readout_body(x_ref, seg_ref, wr_ref, br_ref, o_ref):
    i = pl.program_id(0)
    h = jnp.dot(x_ref[...], wr_ref[...],
                preferred_element_type=jnp.float32) + br_ref[...]
    seg = seg_ref[0]
    onehot = (lax.broadcasted_iota(jnp.int32, (2 * B, _NBLK), 0)
              == seg).astype(jnp.float32)
    g = jnp.dot(onehot, h, preferred_element_type=jnp.float32)

    @pl.when(i == 0)
    def _():
        o_ref[...] = jnp.zeros_like(o_ref)

    o_ref[...] += g


def _tc_readout(nodes, seg, Wr, br):
    nblk = (2 * N) // _NBLK
    return pl.pallas_call(
        _tc_readout_body,
        grid=(nblk,),
        in_specs=[
            pl.BlockSpec((_NBLK, D), lambda i: (i, 0)),
            pl.BlockSpec((1, 1, _NBLK), lambda i: (i, 0, 0)),
            pl.BlockSpec((D, D_READ), lambda i: (0, 0)),
            pl.BlockSpec((1, D_READ), lambda i: (0, 0)),
        ],
        out_specs=pl.BlockSpec((2 * B, D_READ), lambda i: (0, 0)),
        out_shape=jax.ShapeDtypeStruct((2 * B, D_READ), jnp.float32),
    )(nodes, seg, Wr, br)


def kernel(node1, edge1, node2, edge2, Wn, We1, be1, We2, be2, Wk, Wv, Wo, bo,
           ln_w, ln_b, Wr, br,
           seg_g1, inn_seg_i1, inn_idx_j1, out_seg_i1, out_idx_j1,
           seg_g2, inn_seg_i2, inn_idx_j2, out_seg_i2, out_idx_j2):
    f32 = jnp.float32
    zrows = jnp.zeros((RPS, D), f32)
    znorm = jnp.zeros((N,), f32)

    e1_s = _tc_edge(edge1, We1, be1, We2, be2)
    e2_s = _tc_edge(edge2, We1, be1, We2, be2)

    nd1, nd2 = node1, node2
    ee = norm1 = norm2 = None
    for s in range(STEPS):
        P1, K1, V1 = _tc_proj(nd1, Wn[s], Wk[s], Wv[s])
        P2, K2, V2 = _tc_proj(nd2, Wn[s], Wk[s], Wv[s])
        ee, n1p, n2p = _sc_coat1(K1, K2, out_seg_i1, out_seg_i2, znorm)
        m1p = _sc_vscatter(V2, out_seg_i2, out_seg_i1, ee, zrows)
        m2p = _sc_vscatter(V1, out_seg_i1, out_seg_i2, ee, zrows)
        im1p = _sc_mp(P1, e1_s[s], inn_idx_j1, inn_seg_i1, zrows)
        im2p = _sc_mp(P2, e2_s[s], inn_idx_j2, inn_seg_i2, zrows)
        norm1, norm2 = _tc_normred(n1p, n2p)
        bo_r = bo[s].reshape(1, D)
        lw_r = ln_w[s].reshape(1, D)
        lb_r = ln_b[s].reshape(1, D)
        nd1 = _tc_update(nd1, im1p, m1p, norm1.reshape(N, 1),
                         Wo[s], bo_r, lw_r, lb_r)
        nd2 = _tc_update(nd2, im2p, m2p, norm2.reshape(N, 1),
                         Wo[s], bo_r, lw_r, lb_r)

    a1, a2 = _sc_att(out_seg_i1, out_seg_i2, ee, norm1, norm2)

    nodes = jnp.concatenate([nd1, nd2], axis=0)
    seg = jnp.concatenate([seg_g1, seg_g2 + B]).reshape(2 * N // _NBLK, 1,
                                                        _NBLK)
    g = _tc_readout(nodes, seg, Wr, br.reshape(1, D_READ))
    return (g[:B], g[B:], a1[:, None], a2[:, None])
